# Initial kernel scaffold; baseline (speedup 1.0000x reference)
import jax
import jax.numpy as jnp
from jax.experimental import pallas as pl


def kernel(x, W1, b1, W2, b2, W3, b3, W4, b4, codebook):
    raise NotImplementedError("write your pallas kernel here")


# speedup vs baseline: 3.0483x; 3.0483x over previous
import functools

import jax
import jax.numpy as jnp
from jax.experimental import pallas as pl
from jax.experimental.pallas import tpu as pltpu

_N_EMB = 256
_E_DIM = 256
_D_LAT = 8
_HID = 2048

_DN_LT = (((0,), (0,)), ((), ()))
_DN_RT = (((1,), (1,)), ((), ()))
_DN_LT_RT = (((0,), (1,)), ((), ()))


def _body(xt_ref, w1t_ref, b1_ref, w2p_ref, b2p_ref, cb_ref,
          w3p_ref, b3_ref, w4t_ref, b4_ref,
          recont_ref, ze_ref, emb_ref, *, tn):
    f32 = jnp.float32
    xt = xt_ref[...]
    h1 = jnp.maximum(
        jax.lax.dot_general(xt, w1t_ref[...], _DN_LT,
                            preferred_element_type=f32) + b1_ref[...], 0.0)
    za = jnp.dot(h1, w2p_ref[...], preferred_element_type=f32) + b2p_ref[...]
    ze_ref[...] = za.reshape(tn, _D_LAT, _E_DIM)

    cb = cb_ref[...]
    cn = jnp.sum(cb * cb, axis=0, keepdims=True)
    lane = jax.lax.broadcasted_iota(jnp.int32, (tn, _N_EMB), 1)
    e_parts = []
    for d in range(_D_LAT):
        zd = za[:, d * _E_DIM:(d + 1) * _E_DIM]
        zn = jnp.sum(zd * zd, axis=1, keepdims=True)
        d2 = zn - 2.0 * jnp.dot(zd, cb, preferred_element_type=f32) + cn
        idx = jnp.argmin(d2, axis=1)
        oh = (lane == idx[:, None]).astype(f32)
        e_parts.append(jax.lax.dot_general(
            oh, cb, _DN_RT, preferred_element_type=f32))
    e_cat = jnp.concatenate(e_parts, axis=1)
    emb_ref[...] = e_cat.reshape(tn, _D_LAT, _E_DIM)

    h3 = jnp.maximum(
        jnp.dot(e_cat, w3p_ref[...], preferred_element_type=f32)
        + b3_ref[...], 0.0)
    h4t = jax.lax.dot_general(w4t_ref[...], h3, _DN_LT_RT,
                              preferred_element_type=f32) + b4_ref[...]
    recont_ref[...] = jnp.tanh(jax.nn.sigmoid(h4t))


def kernel(x, W1, b1, W2, b2, W3, b3, W4, b4, codebook):
    n = x.shape[0]
    tn = 512
    grid = n // tn

    xt = x.T
    W1T = W1.T
    W2P = jnp.transpose(W2.reshape(_E_DIM, _D_LAT, 400), (1, 0, 2)) \
             .reshape(_HID, 400).T
    b2P = b2.reshape(_E_DIM, _D_LAT).T.reshape(1, _HID)
    W3P = jnp.transpose(W3.reshape(400, _E_DIM, _D_LAT), (2, 1, 0)) \
             .reshape(_HID, 400)
    W4T = W4.T

    b1r = b1.reshape(1, 400)
    b3r = b3.reshape(1, 400)
    b4c = b4.reshape(800, 1)

    full = lambda a: pl.BlockSpec(a.shape, lambda i: (0,) * a.ndim)
    col = lambda h: pl.BlockSpec((h, tn), lambda i: (0, i))
    r3 = pl.BlockSpec((tn, _D_LAT, _E_DIM), lambda i: (i, 0, 0))

    recont, za3, emb3 = pl.pallas_call(
        functools.partial(_body, tn=tn),
        grid=(grid,),
        in_specs=[
            col(800),
            full(W1T), full(b1r), full(W2P), full(b2P), full(codebook),
            full(W3P), full(b3r), full(W4T), full(b4c),
        ],
        out_specs=[col(800), r3, r3],
        out_shape=[
            jax.ShapeDtypeStruct((800, n), jnp.float32),
            jax.ShapeDtypeStruct((n, _D_LAT, _E_DIM), jnp.float32),
            jax.ShapeDtypeStruct((n, _D_LAT, _E_DIM), jnp.float32),
        ],
        compiler_params=pltpu.CompilerParams(
            dimension_semantics=("arbitrary",)),
    )(xt, W1T, b1r, W2P, b2P, codebook, W3P, b3r, W4T, b4c)

    recon = recont.T
    z_e = jnp.transpose(za3, (0, 2, 1))
    emb = jnp.transpose(emb3, (0, 2, 1))
    return recon, z_e, emb
